# Initial kernel scaffold; baseline (speedup 1.0000x reference)
import jax
import jax.numpy as jnp
from jax.experimental import pallas as pl


def kernel(out_logits, out_bbox, target_sizes):
    raise NotImplementedError("write your pallas kernel here")


# speedup vs baseline: 20.3579x; 20.3579x over previous
import functools

import jax
import jax.numpy as jnp
from jax import lax
from jax.experimental import pallas as pl
from jax.experimental.pallas import tpu as pltpu

_NEG = float("-inf")
_BIG = 1 << 30
NCAND = 128


def _rowmax_body(x_ref, bm_ref):
    bm_ref[...] = jnp.max(x_ref[...], axis=2)[:, None, :]


def _select_body(n_iter, x_ref, vals_ref, pos_ref):
    x = x_ref[...]
    B, L = x.shape
    iota = lax.broadcasted_iota(jnp.int32, (B, L), 1)
    slot = lax.broadcasted_iota(jnp.int32, (B, NCAND), 1)

    def step(it, carry):
        x_c, vals, pos = carry
        m = jnp.max(x_c, axis=1, keepdims=True)
        p = jnp.min(jnp.where(x_c == m, iota, _BIG), axis=1, keepdims=True)
        vals = jnp.where(slot == it, m, vals)
        pos = jnp.where(slot == it, p, pos)
        x_c = jnp.where(iota == p, _NEG, x_c)
        return x_c, vals, pos

    vals0 = jnp.full((B, NCAND), _NEG, jnp.float32)
    pos0 = jnp.zeros((B, NCAND), jnp.int32)
    _, vals, pos = lax.fori_loop(0, n_iter, step, (x, vals0, pos0))
    vals_ref[...] = vals
    pos_ref[...] = pos


def _topk_rows(x, n_iter):
    B, L = x.shape
    return pl.pallas_call(
        functools.partial(_select_body, n_iter),
        in_specs=[pl.BlockSpec((B, L), lambda: (0, 0))],
        out_specs=[
            pl.BlockSpec((B, NCAND), lambda: (0, 0)),
            pl.BlockSpec((B, NCAND), lambda: (0, 0)),
        ],
        out_shape=[
            jax.ShapeDtypeStruct((B, NCAND), jnp.float32),
            jax.ShapeDtypeStruct((B, NCAND), jnp.int32),
        ],
    )(x)


def kernel(out_logits, out_bbox, target_sizes):
    B, N, C = out_logits.shape

    bm = pl.pallas_call(
        _rowmax_body,
        grid=(B,),
        in_specs=[pl.BlockSpec((1, N, C), lambda b: (b, 0, 0))],
        out_specs=pl.BlockSpec((1, 1, N), lambda b: (b, 0, 0)),
        out_shape=jax.ShapeDtypeStruct((B, 1, N), jnp.float32),
    )(out_logits)
    bm = bm.reshape(B, N)

    _, bids = _topk_rows(bm, NCAND)
    bids = jnp.sort(bids, axis=1)

    g = jnp.take_along_axis(out_logits, bids[:, :, None], axis=1)

    vals, pos = _topk_rows(g.reshape(B, NCAND * C), 100)
    vals = vals[:, :100]
    pos = pos[:, :100]
    cand = pos // C
    labels = pos % C
    box_id = jnp.take_along_axis(bids, cand, axis=1)

    scores = jax.nn.sigmoid(vals)
    bb = jnp.take_along_axis(out_bbox, box_id[:, :, None], axis=1)
    x_c, y_c, w, h = bb[..., 0], bb[..., 1], bb[..., 2], bb[..., 3]
    boxes = jnp.stack(
        [x_c - 0.5 * w, y_c - 0.5 * h, x_c + 0.5 * w, y_c + 0.5 * h], axis=-1
    )
    img_h = target_sizes[:, 0].astype(jnp.float32)
    img_w = target_sizes[:, 1].astype(jnp.float32)
    scale = jnp.stack([img_w, img_h, img_w, img_h], axis=1)
    boxes = boxes * scale[:, None, :]
    return scores, labels, boxes
